# Initial kernel scaffold; baseline (speedup 1.0000x reference)
import jax
import jax.numpy as jnp
from jax.experimental import pallas as pl


def kernel(from_up, from_down, neighbors, W1, b1, W2, b2):
    raise NotImplementedError("write your pallas kernel here")


# speedup vs baseline: 3.5937x; 3.5937x over previous
import functools

import jax
import jax.numpy as jnp
from jax import lax
from jax.experimental import pallas as pl
from jax.experimental.pallas import tpu as pltpu
from jax.experimental.pallas import tpu_sc as plsc

N = 50000
CI = 128
CO = 128
K = 6

NC = 2
NS = 16
NWORK = NC * NS
NPW = 1568
NPAD = NWORK * NPW
CHUNK = 112
NCHUNK = NPW // CHUNK
BN = 1024
GRID = NPAD // BN
EPS = 1e-5


def _proj1_body(x_ref, w_ref, b_ref, y_ref, s_ref):
    xb = x_ref[...]
    for j in range(K + 1):
        w = w_ref[:, j * CO:(j + 1) * CO]
        r = lax.dot_general(xb, w, (((0,), (0,)), ((), ())),
                            preferred_element_type=jnp.float32)
        if j == 0:
            s_ref[...] = r + b_ref[...]
        else:
            y_ref[j - 1, :, :] = r


def _proj1(x_pad, w1cat, b1r):
    return pl.pallas_call(
        _proj1_body,
        grid=(GRID,),
        in_specs=[
            pl.BlockSpec((CI, BN), lambda i: (0, i)),
            pl.BlockSpec((CI, (K + 1) * CO), lambda i: (0, 0)),
            pl.BlockSpec((1, CO), lambda i: (0, 0)),
        ],
        out_specs=[
            pl.BlockSpec((K, BN, CO), lambda i: (0, i, 0)),
            pl.BlockSpec((BN, CO), lambda i: (i, 0)),
        ],
        out_shape=[
            jax.ShapeDtypeStruct((K, NPAD, CO), jnp.float32),
            jax.ShapeDtypeStruct((NPAD, CO), jnp.float32),
        ],
    )(x_pad, w1cat, b1r)


def _proj2_body(g1_ref, s1_ref, xd_ref, wa_ref, wb_ref, b_ref, y_ref, s_ref):
    x1 = g1_ref[...] + s1_ref[...]
    xd = xd_ref[...]
    for j in range(K + 1):
        wa = wa_ref[:, j * CO:(j + 1) * CO]
        wb = wb_ref[:, j * CO:(j + 1) * CO]
        r = lax.dot_general(x1, wa, (((1,), (0,)), ((), ())),
                            preferred_element_type=jnp.float32)
        r = r + lax.dot_general(xd, wb, (((0,), (0,)), ((), ())),
                                preferred_element_type=jnp.float32)
        if j == 0:
            s_ref[...] = r + b_ref[...]
        else:
            y_ref[j - 1, :, :] = r


def _proj2(g1, s1, xd_pad, w2a, w2b, b2r):
    return pl.pallas_call(
        _proj2_body,
        grid=(GRID,),
        in_specs=[
            pl.BlockSpec((BN, CO), lambda i: (i, 0)),
            pl.BlockSpec((BN, CO), lambda i: (i, 0)),
            pl.BlockSpec((CO, BN), lambda i: (0, i)),
            pl.BlockSpec((CO, (K + 1) * CO), lambda i: (0, 0)),
            pl.BlockSpec((CO, (K + 1) * CO), lambda i: (0, 0)),
            pl.BlockSpec((1, CO), lambda i: (0, 0)),
        ],
        out_specs=[
            pl.BlockSpec((K, BN, CO), lambda i: (0, i, 0)),
            pl.BlockSpec((BN, CO), lambda i: (i, 0)),
        ],
        out_shape=[
            jax.ShapeDtypeStruct((K, NPAD, CO), jnp.float32),
            jax.ShapeDtypeStruct((NPAD, CO), jnp.float32),
        ],
    )(g1, s1, xd_pad, w2a, w2b, b2r)


def _gather_body(table_hbm, idx_hbm, out_hbm, idx_v, buf, obuf, sem):
    wid = lax.axis_index("s") * NC + lax.axis_index("c")

    def chunk_body(g, carry):
        chunkid = wid * NCHUNK + g
        pltpu.sync_copy(idx_hbm.at[chunkid], idx_v)
        cps = [pltpu.async_copy(table_hbm.at[idx_v.at[t]], buf.at[t], sem)
               for t in range(K)]
        for cp in cps:
            cp.wait()

        def row_body(r, c2):
            for k in range(CO // 16):
                sl = pl.ds(k * 16, 16)
                acc = buf[0, r, sl]
                for t in range(1, K):
                    acc = acc + buf[t, r, sl]
                obuf[r, sl] = acc
            return c2

        lax.fori_loop(0, CHUNK, row_body, 0)
        pltpu.sync_copy(obuf, out_hbm.at[pl.ds(chunkid * CHUNK, CHUNK)])
        return carry

    lax.fori_loop(0, NCHUNK, chunk_body, 0)


_gather_sum = functools.partial(
    pl.kernel,
    mesh=plsc.VectorSubcoreMesh(core_axis_name="c", subcore_axis_name="s"),
    out_type=jax.ShapeDtypeStruct((NPAD, CO), jnp.float32),
    scratch_types=[
        pltpu.VMEM((K, CHUNK), jnp.int32),
        pltpu.VMEM((K, CHUNK, CO), jnp.float32),
        pltpu.VMEM((CHUNK, CO), jnp.float32),
        pltpu.SemaphoreType.DMA,
    ],
)(_gather_body)


def _stats_body(g_ref, s_ref, o_ref):
    i = pl.program_id(0)

    @pl.when(i == 0)
    def _():
        o_ref[...] = jnp.zeros_like(o_ref)

    z = g_ref[...] + s_ref[...]
    gid = i * BN + lax.broadcasted_iota(jnp.int32, (BN, CO), 0)
    zm = jnp.where(gid < N, z, 0.0)
    s = jnp.sum(zm, axis=0)
    q = jnp.sum(zm * zm, axis=0)
    o_ref[0, :] += s
    o_ref[1, :] += q


def _stats(g2, s2):
    return pl.pallas_call(
        _stats_body,
        grid=(GRID,),
        in_specs=[
            pl.BlockSpec((BN, CO), lambda i: (i, 0)),
            pl.BlockSpec((BN, CO), lambda i: (i, 0)),
        ],
        out_specs=pl.BlockSpec((8, CO), lambda i: (0, 0)),
        out_shape=jax.ShapeDtypeStruct((8, CO), jnp.float32),
    )(g2, s2)


def _norm_body(g_ref, s_ref, st_ref, o_ref):
    z = g_ref[...] + s_ref[...]
    mean = st_ref[0, :] * (1.0 / N)
    var = st_ref[1, :] * (1.0 / N) - mean * mean
    inv = lax.rsqrt(var + EPS)
    zn = (z - mean[None, :]) * inv[None, :]
    o_ref[...] = zn.T


def _norm(g2, s2, st):
    return pl.pallas_call(
        _norm_body,
        grid=(GRID,),
        in_specs=[
            pl.BlockSpec((BN, CO), lambda i: (i, 0)),
            pl.BlockSpec((BN, CO), lambda i: (i, 0)),
            pl.BlockSpec((8, CO), lambda i: (0, 0)),
        ],
        out_specs=pl.BlockSpec((CO, BN), lambda i: (0, i)),
        out_shape=jax.ShapeDtypeStruct((CO, NPAD), jnp.float32),
    )(g2, s2, st)


def kernel(from_up, from_down, neighbors, W1, b1, W2, b2):
    f32 = jnp.float32
    xu = jnp.pad(from_up[0], ((0, 0), (0, NPAD - N)))
    xd = jnp.pad(from_down[0], ((0, 0), (0, NPAD - N)))

    w1cat = W1.transpose(1, 2, 0).reshape(CI, (K + 1) * CO).astype(f32)
    w2a = W2[:, :CO, :].transpose(1, 2, 0).reshape(CO, (K + 1) * CO).astype(f32)
    w2b = W2[:, CO:, :].transpose(1, 2, 0).reshape(CO, (K + 1) * CO).astype(f32)
    b1r = b1[None, :].astype(f32)
    b2r = b2[None, :].astype(f32)

    nbr_pad = jnp.concatenate(
        [neighbors.astype(jnp.int32),
         jnp.zeros((NPAD - N, K), jnp.int32)], axis=0)
    adj = nbr_pad + (jnp.arange(K, dtype=jnp.int32) * NPAD)[None, :]
    idxarr = adj.reshape(NWORK * NCHUNK, CHUNK, K).transpose(0, 2, 1)

    y1, s1 = _proj1(xu, w1cat, b1r)
    g1 = _gather_sum(y1.reshape(K * NPAD, CO), idxarr)
    y2, s2 = _proj2(g1, s1, xd, w2a, w2b, b2r)
    g2 = _gather_sum(y2.reshape(K * NPAD, CO), idxarr)
    st = _stats(g2, s2)
    outp = _norm(g2, s2, st)
    return outp[:, :N][None]


# speedup vs baseline: 3.7869x; 1.0537x over previous
import functools

import jax
import jax.numpy as jnp
from jax import lax
from jax.experimental import pallas as pl
from jax.experimental.pallas import tpu as pltpu
from jax.experimental.pallas import tpu_sc as plsc

N = 50000
CI = 128
CO = 128
K = 6

NC = 2
NS = 16
NWORK = NC * NS
NPW = 1568
NPAD = NWORK * NPW
CHUNK = 112
NCHUNK = NPW // CHUNK
BN = 1024
GRID = NPAD // BN
EPS = 1e-5


def _proj1_body(x_ref, w_ref, b_ref, y_ref, s_ref):
    xb = x_ref[...]
    for j in range(K + 1):
        w = w_ref[:, j * CO:(j + 1) * CO]
        r = lax.dot_general(xb, w, (((0,), (0,)), ((), ())),
                            preferred_element_type=jnp.float32)
        if j == 0:
            s_ref[...] = r + b_ref[...]
        else:
            y_ref[j - 1, :, :] = r


def _proj1(x_pad, w1cat, b1r):
    return pl.pallas_call(
        _proj1_body,
        grid=(GRID,),
        in_specs=[
            pl.BlockSpec((CI, BN), lambda i: (0, i)),
            pl.BlockSpec((CI, (K + 1) * CO), lambda i: (0, 0)),
            pl.BlockSpec((1, CO), lambda i: (0, 0)),
        ],
        out_specs=[
            pl.BlockSpec((K, BN, CO), lambda i: (0, i, 0)),
            pl.BlockSpec((BN, CO), lambda i: (i, 0)),
        ],
        out_shape=[
            jax.ShapeDtypeStruct((K, NPAD, CO), jnp.float32),
            jax.ShapeDtypeStruct((NPAD, CO), jnp.float32),
        ],
    )(x_pad, w1cat, b1r)


def _proj2_body(g1_ref, s1_ref, xd_ref, wa_ref, wb_ref, b_ref, y_ref, s_ref):
    x1 = g1_ref[...] + s1_ref[...]
    xd = xd_ref[...]
    for j in range(K + 1):
        wa = wa_ref[:, j * CO:(j + 1) * CO]
        wb = wb_ref[:, j * CO:(j + 1) * CO]
        r = lax.dot_general(x1, wa, (((1,), (0,)), ((), ())),
                            preferred_element_type=jnp.float32)
        r = r + lax.dot_general(xd, wb, (((0,), (0,)), ((), ())),
                                preferred_element_type=jnp.float32)
        if j == 0:
            s_ref[...] = r + b_ref[...]
        else:
            y_ref[j - 1, :, :] = r


def _proj2(g1, s1, xd_pad, w2a, w2b, b2r):
    return pl.pallas_call(
        _proj2_body,
        grid=(GRID,),
        in_specs=[
            pl.BlockSpec((BN, CO), lambda i: (i, 0)),
            pl.BlockSpec((BN, CO), lambda i: (i, 0)),
            pl.BlockSpec((CO, BN), lambda i: (0, i)),
            pl.BlockSpec((CO, (K + 1) * CO), lambda i: (0, 0)),
            pl.BlockSpec((CO, (K + 1) * CO), lambda i: (0, 0)),
            pl.BlockSpec((1, CO), lambda i: (0, 0)),
        ],
        out_specs=[
            pl.BlockSpec((K, BN, CO), lambda i: (0, i, 0)),
            pl.BlockSpec((BN, CO), lambda i: (i, 0)),
        ],
        out_shape=[
            jax.ShapeDtypeStruct((K, NPAD, CO), jnp.float32),
            jax.ShapeDtypeStruct((NPAD, CO), jnp.float32),
        ],
    )(g1, s1, xd_pad, w2a, w2b, b2r)


def _gather_body(table_hbm, idx_hbm, out_hbm, idx_v, buf, obuf, sem):
    wid = lax.axis_index("s") * NC + lax.axis_index("c")

    def chunk_body(g, carry):
        chunkid = wid * NCHUNK + g
        pltpu.sync_copy(idx_hbm.at[chunkid], idx_v)
        cps = [pltpu.async_copy(table_hbm.at[idx_v.at[t]], buf.at[t], sem)
               for t in range(K)]
        for cp in cps:
            cp.wait()

        def row_body(r, c2):
            for k in range(CO // 16):
                sl = pl.ds(k * 16, 16)
                acc = buf[0, r, sl]
                for t in range(1, K):
                    acc = acc + buf[t, r, sl]
                obuf[r, sl] = acc
            return c2

        lax.fori_loop(0, CHUNK, row_body, 0)
        pltpu.sync_copy(obuf, out_hbm.at[pl.ds(chunkid * CHUNK, CHUNK)])
        return carry

    lax.fori_loop(0, NCHUNK, chunk_body, 0)


_gather_sum = functools.partial(
    pl.kernel,
    mesh=plsc.VectorSubcoreMesh(core_axis_name="c", subcore_axis_name="s"),
    out_type=jax.ShapeDtypeStruct((NPAD, CO), jnp.float32),
    scratch_types=[
        pltpu.VMEM((K, CHUNK), jnp.int32),
        pltpu.VMEM((K, CHUNK, CO), jnp.float32),
        pltpu.VMEM((CHUNK, CO), jnp.float32),
        pltpu.SemaphoreType.DMA,
    ],
)(_gather_body)


def _stats_body(g_ref, s_ref, o_ref):
    i = pl.program_id(0)

    @pl.when(i == 0)
    def _():
        o_ref[...] = jnp.zeros_like(o_ref)

    z = g_ref[...] + s_ref[...]
    gid = i * BN + lax.broadcasted_iota(jnp.int32, (BN, CO), 0)
    zm = jnp.where(gid < N, z, 0.0)
    s = jnp.sum(zm, axis=0)
    q = jnp.sum(zm * zm, axis=0)
    o_ref[0, :] += s
    o_ref[1, :] += q


def _stats(g2, s2):
    return pl.pallas_call(
        _stats_body,
        grid=(GRID,),
        in_specs=[
            pl.BlockSpec((BN, CO), lambda i: (i, 0)),
            pl.BlockSpec((BN, CO), lambda i: (i, 0)),
        ],
        out_specs=pl.BlockSpec((8, CO), lambda i: (0, 0)),
        out_shape=jax.ShapeDtypeStruct((8, CO), jnp.float32),
    )(g2, s2)


def _norm_body(g_ref, s_ref, st_ref, o_ref):
    z = g_ref[...] + s_ref[...]
    mean = st_ref[0, :] * (1.0 / N)
    var = st_ref[1, :] * (1.0 / N) - mean * mean
    inv = lax.rsqrt(var + EPS)
    zn = (z - mean[None, :]) * inv[None, :]
    o_ref[...] = zn.T[None]


def _norm(g2, s2, st):
    return pl.pallas_call(
        _norm_body,
        grid=(GRID,),
        in_specs=[
            pl.BlockSpec((BN, CO), lambda i: (i, 0)),
            pl.BlockSpec((BN, CO), lambda i: (i, 0)),
            pl.BlockSpec((8, CO), lambda i: (0, 0)),
        ],
        out_specs=pl.BlockSpec((1, CO, BN), lambda i: (0, 0, i)),
        out_shape=jax.ShapeDtypeStruct((1, CO, N), jnp.float32),
    )(g2, s2, st)


def kernel(from_up, from_down, neighbors, W1, b1, W2, b2):
    f32 = jnp.float32
    xu = from_up[0]
    xd = from_down[0]

    w1cat = W1.transpose(1, 2, 0).reshape(CI, (K + 1) * CO).astype(f32)
    w2a = W2[:, :CO, :].transpose(1, 2, 0).reshape(CO, (K + 1) * CO).astype(f32)
    w2b = W2[:, CO:, :].transpose(1, 2, 0).reshape(CO, (K + 1) * CO).astype(f32)
    b1r = b1[None, :].astype(f32)
    b2r = b2[None, :].astype(f32)

    nbr_pad = jnp.concatenate(
        [neighbors.astype(jnp.int32),
         jnp.zeros((NPAD - N, K), jnp.int32)], axis=0)
    adj = nbr_pad + (jnp.arange(K, dtype=jnp.int32) * NPAD)[None, :]
    idxarr = adj.reshape(NWORK * NCHUNK, CHUNK, K).transpose(0, 2, 1)

    y1, s1 = _proj1(xu, w1cat, b1r)
    g1 = _gather_sum(y1.reshape(K * NPAD, CO), idxarr)
    y2, s2 = _proj2(g1, s1, xd, w2a, w2b, b2r)
    g2 = _gather_sum(y2.reshape(K * NPAD, CO), idxarr)
    st = _stats(g2, s2)
    return _norm(g2, s2, st)


# speedup vs baseline: 4.5753x; 1.2082x over previous
import functools

import jax
import jax.numpy as jnp
from jax import lax
from jax.experimental import pallas as pl
from jax.experimental.pallas import tpu as pltpu
from jax.experimental.pallas import tpu_sc as plsc

N = 50000
CI = 128
CO = 128
K = 6

NC = 2
NS = 16
NWORK = NC * NS
NPW = 1568
NPAD = NWORK * NPW
CHUNK = 56
NCHUNK = NPW // CHUNK
BN = 1024
GRID = NPAD // BN
EPS = 1e-5


def _proj1_body(x_ref, w_ref, b_ref, y_ref, s_ref):
    xb = x_ref[...]
    for j in range(K + 1):
        w = w_ref[:, j * CO:(j + 1) * CO]
        r = lax.dot_general(xb, w, (((0,), (0,)), ((), ())),
                            preferred_element_type=jnp.float32)
        if j == 0:
            s_ref[...] = r + b_ref[...]
        else:
            y_ref[j - 1, :, :] = r


def _proj1(x_pad, w1cat, b1r):
    return pl.pallas_call(
        _proj1_body,
        grid=(GRID,),
        in_specs=[
            pl.BlockSpec((CI, BN), lambda i: (0, i)),
            pl.BlockSpec((CI, (K + 1) * CO), lambda i: (0, 0)),
            pl.BlockSpec((1, CO), lambda i: (0, 0)),
        ],
        out_specs=[
            pl.BlockSpec((K, BN, CO), lambda i: (0, i, 0)),
            pl.BlockSpec((BN, CO), lambda i: (i, 0)),
        ],
        out_shape=[
            jax.ShapeDtypeStruct((K, NPAD, CO), jnp.float32),
            jax.ShapeDtypeStruct((NPAD, CO), jnp.float32),
        ],
    )(x_pad, w1cat, b1r)


def _proj2_body(g1_ref, s1_ref, xd_ref, wa_ref, wb_ref, b_ref, y_ref, s_ref):
    x1 = g1_ref[...] + s1_ref[...]
    xd = xd_ref[...]
    for j in range(K + 1):
        wa = wa_ref[:, j * CO:(j + 1) * CO]
        wb = wb_ref[:, j * CO:(j + 1) * CO]
        r = lax.dot_general(x1, wa, (((1,), (0,)), ((), ())),
                            preferred_element_type=jnp.float32)
        r = r + lax.dot_general(xd, wb, (((0,), (0,)), ((), ())),
                                preferred_element_type=jnp.float32)
        if j == 0:
            s_ref[...] = r + b_ref[...]
        else:
            y_ref[j - 1, :, :] = r


def _proj2(g1, s1, xd_pad, w2a, w2b, b2r):
    return pl.pallas_call(
        _proj2_body,
        grid=(GRID,),
        in_specs=[
            pl.BlockSpec((BN, CO), lambda i: (i, 0)),
            pl.BlockSpec((BN, CO), lambda i: (i, 0)),
            pl.BlockSpec((CO, BN), lambda i: (0, i)),
            pl.BlockSpec((CO, (K + 1) * CO), lambda i: (0, 0)),
            pl.BlockSpec((CO, (K + 1) * CO), lambda i: (0, 0)),
            pl.BlockSpec((1, CO), lambda i: (0, 0)),
        ],
        out_specs=[
            pl.BlockSpec((K, BN, CO), lambda i: (0, i, 0)),
            pl.BlockSpec((BN, CO), lambda i: (i, 0)),
        ],
        out_shape=[
            jax.ShapeDtypeStruct((K, NPAD, CO), jnp.float32),
            jax.ShapeDtypeStruct((NPAD, CO), jnp.float32),
        ],
    )(g1, s1, xd_pad, w2a, w2b, b2r)


def _gather_body(table_hbm, idx_hbm, out_hbm, idx_v, buf, obuf,
                 semg0, semg1, sems0, sems1):
    wid = lax.axis_index("s") * NC + lax.axis_index("c")
    base_chunk = wid * NCHUNK
    semg = (semg0, semg1)
    sems = (sems0, sems1)

    def fire(b, g):
        pltpu.sync_copy(idx_hbm.at[base_chunk + g], idx_v.at[b])
        for t in range(K):
            pltpu.async_copy(table_hbm.at[idx_v.at[b, t]], buf.at[b, t],
                             semg[b])

    def wait_gathers(b):
        for t in range(K):
            pltpu.make_async_copy(table_hbm.at[pl.ds(0, CHUNK)],
                                  buf.at[b, t], semg[b]).wait()

    def accumulate(b):
        def row_body(r, c2):
            for k in range(CO // 16):
                sl = pl.ds(k * 16, 16)
                acc = buf[b, 0, r, sl]
                for t in range(1, K):
                    acc = acc + buf[b, t, r, sl]
                obuf[b, r, sl] = acc
            return c2

        lax.fori_loop(0, CHUNK, row_body, 0)

    def consume(b, i):
        g = 2 * i + b
        wait_gathers(b)

        @pl.when(i > 0)
        def _():
            pltpu.make_async_copy(obuf.at[b], out_hbm.at[pl.ds(0, CHUNK)],
                                  sems[b]).wait()

        accumulate(b)
        pltpu.async_copy(obuf.at[b],
                         out_hbm.at[pl.ds((base_chunk + g) * CHUNK, CHUNK)],
                         sems[b])

        @pl.when(i < NCHUNK // 2 - 1)
        def _():
            fire(b, g + 2)

    fire(0, 0)
    fire(1, 1)

    def pair_body(i, carry):
        consume(0, i)
        consume(1, i)
        return carry

    lax.fori_loop(0, NCHUNK // 2, pair_body, 0)
    for b in range(2):
        pltpu.make_async_copy(obuf.at[b], out_hbm.at[pl.ds(0, CHUNK)],
                              sems[b]).wait()


_gather_sum = functools.partial(
    pl.kernel,
    mesh=plsc.VectorSubcoreMesh(core_axis_name="c", subcore_axis_name="s"),
    out_type=jax.ShapeDtypeStruct((NPAD, CO), jnp.float32),
    scratch_types=[
        pltpu.VMEM((2, K, CHUNK), jnp.int32),
        pltpu.VMEM((2, K, CHUNK, CO), jnp.float32),
        pltpu.VMEM((2, CHUNK, CO), jnp.float32),
        pltpu.SemaphoreType.DMA,
        pltpu.SemaphoreType.DMA,
        pltpu.SemaphoreType.DMA,
        pltpu.SemaphoreType.DMA,
    ],
)(_gather_body)


def _stats_body(g_ref, s_ref, o_ref):
    i = pl.program_id(0)

    @pl.when(i == 0)
    def _():
        o_ref[...] = jnp.zeros_like(o_ref)

    z = g_ref[...] + s_ref[...]
    gid = i * BN + lax.broadcasted_iota(jnp.int32, (BN, CO), 0)
    zm = jnp.where(gid < N, z, 0.0)
    s = jnp.sum(zm, axis=0)
    q = jnp.sum(zm * zm, axis=0)
    o_ref[0, :] += s
    o_ref[1, :] += q


def _stats(g2, s2):
    return pl.pallas_call(
        _stats_body,
        grid=(GRID,),
        in_specs=[
            pl.BlockSpec((BN, CO), lambda i: (i, 0)),
            pl.BlockSpec((BN, CO), lambda i: (i, 0)),
        ],
        out_specs=pl.BlockSpec((8, CO), lambda i: (0, 0)),
        out_shape=jax.ShapeDtypeStruct((8, CO), jnp.float32),
    )(g2, s2)


def _norm_body(g_ref, s_ref, st_ref, o_ref):
    z = g_ref[...] + s_ref[...]
    mean = st_ref[0, :] * (1.0 / N)
    var = st_ref[1, :] * (1.0 / N) - mean * mean
    inv = lax.rsqrt(var + EPS)
    zn = (z - mean[None, :]) * inv[None, :]
    o_ref[...] = zn.T[None]


def _norm(g2, s2, st):
    return pl.pallas_call(
        _norm_body,
        grid=(GRID,),
        in_specs=[
            pl.BlockSpec((BN, CO), lambda i: (i, 0)),
            pl.BlockSpec((BN, CO), lambda i: (i, 0)),
            pl.BlockSpec((8, CO), lambda i: (0, 0)),
        ],
        out_specs=pl.BlockSpec((1, CO, BN), lambda i: (0, 0, i)),
        out_shape=jax.ShapeDtypeStruct((1, CO, N), jnp.float32),
    )(g2, s2, st)


def kernel(from_up, from_down, neighbors, W1, b1, W2, b2):
    f32 = jnp.float32
    xu = from_up[0]
    xd = from_down[0]

    w1cat = W1.transpose(1, 2, 0).reshape(CI, (K + 1) * CO).astype(f32)
    w2a = W2[:, :CO, :].transpose(1, 2, 0).reshape(CO, (K + 1) * CO).astype(f32)
    w2b = W2[:, CO:, :].transpose(1, 2, 0).reshape(CO, (K + 1) * CO).astype(f32)
    b1r = b1[None, :].astype(f32)
    b2r = b2[None, :].astype(f32)

    nbr_pad = jnp.concatenate(
        [neighbors.astype(jnp.int32),
         jnp.zeros((NPAD - N, K), jnp.int32)], axis=0)
    adj = nbr_pad + (jnp.arange(K, dtype=jnp.int32) * NPAD)[None, :]
    idxarr = adj.reshape(NWORK * NCHUNK, CHUNK, K).transpose(0, 2, 1)

    y1, s1 = _proj1(xu, w1cat, b1r)
    g1 = _gather_sum(y1.reshape(K * NPAD, CO), idxarr)
    y2, s2 = _proj2(g1, s1, xd, w2a, w2b, b2r)
    g2 = _gather_sum(y2.reshape(K * NPAD, CO), idxarr)
    st = _stats(g2, s2)
    return _norm(g2, s2, st)
